# baseline (speedup 1.0000x reference)
import functools

import jax
import jax.numpy as jnp
from jax import lax
from jax.experimental import pallas as pl
from jax.experimental.pallas import tpu as pltpu
from jax.experimental.pallas import tpu_sc as plsc

N_ROWS = 16384
N_COLS = 1024
K = 100
L = 16
NG = (K + L - 1) // L
KPAD = NG * L
NC = 2
NS = 16
NW = NC * NS
ROWS_PER_W = N_ROWS // NW
R = 32
NCHUNK = ROWS_PER_W // R

_mesh = plsc.VectorSubcoreMesh(core_axis_name="c", subcore_axis_name="s")


@functools.partial(
    pl.kernel,
    out_type=jax.ShapeDtypeStruct((N_ROWS * K,), jnp.float32),

    mesh=_mesh,
    scratch_types=[
        pltpu.VMEM((KPAD,), jnp.int32),
        pltpu.VMEM((R * N_COLS,), jnp.float32),
        pltpu.VMEM((R * K,), jnp.float32),
        pltpu.SemaphoreType.DMA,
    ],
    compiler_params=pltpu.CompilerParams(
        use_tc_tiling_on_sc=False, needs_layout_passes=False
    ),
)
def _sc_gather(x_hbm, cols_hbm, out_hbm, idx_v, in_v, out_v, sem):
    wid = lax.axis_index("s") * NC + lax.axis_index("c")
    base_row = wid * ROWS_PER_W

    pltpu.async_copy(cols_hbm, idx_v, sem).wait()
    col_vecs = [idx_v[pl.ds(j * L, L)] for j in range(NG)]
    iota = lax.iota(jnp.int32, L)
    out_off = [j * L + iota for j in range(NG)]
    tail_mask = (NG - 1) * L + iota < K

    @pl.loop(0, NCHUNK)
    def _chunk(g):
        row0 = base_row + g * R
        pltpu.async_copy(
            x_hbm.at[pl.ds(row0 * N_COLS, R * N_COLS)], in_v, sem
        ).wait()

        @pl.loop(0, R)
        def _row(r):
            rbase = r * N_COLS
            rk = r * K
            for j in range(NG):
                vals = plsc.load_gather(in_v, [rbase + col_vecs[j]])
                plsc.store_scatter(
                    out_v,
                    [rk + out_off[j]],
                    vals,
                    mask=tail_mask if j == NG - 1 else None,
                )

        pltpu.async_copy(out_v, out_hbm.at[pl.ds(row0 * K, R * K)], sem).wait()


def kernel(x, column_indices):
    cols = jnp.asarray(column_indices, jnp.int32)
    cols_padded = jnp.concatenate([cols, jnp.zeros((KPAD - K,), jnp.int32)])
    out_flat = _sc_gather(x.reshape(-1), cols_padded)
    return out_flat.reshape(N_ROWS, K)


# speedup vs baseline: 1.1302x; 1.1302x over previous
import functools

import jax
import jax.numpy as jnp
from jax import lax
from jax.experimental import pallas as pl
from jax.experimental.pallas import tpu as pltpu
from jax.experimental.pallas import tpu_sc as plsc

N_ROWS = 16384
N_COLS = 1024
K = 100
L = 16
NG = (K + L - 1) // L
KPAD = NG * L
NC = 2
NS = 16
NW = NC * NS
ROWS_PER_W = N_ROWS // NW
R = 32
NCHUNK = ROWS_PER_W // R
NBUF = 2

_mesh = plsc.VectorSubcoreMesh(core_axis_name="c", subcore_axis_name="s")


@functools.partial(
    pl.kernel,
    out_type=jax.ShapeDtypeStruct((N_ROWS * K,), jnp.float32),
    mesh=_mesh,
    scratch_types=[
        pltpu.VMEM((KPAD,), jnp.int32),
        [pltpu.VMEM((R * N_COLS,), jnp.float32) for _ in range(NBUF)],
        [pltpu.VMEM((R * K,), jnp.float32) for _ in range(NBUF)],
        [pltpu.SemaphoreType.DMA for _ in range(NBUF)],
        [pltpu.SemaphoreType.DMA for _ in range(NBUF)],
    ],
    compiler_params=pltpu.CompilerParams(
        use_tc_tiling_on_sc=False, needs_layout_passes=False
    ),
)
def _sc_gather(x_hbm, cols_hbm, out_hbm, idx_v, in_bufs, out_bufs, isems, osems):
    wid = lax.axis_index("s") * NC + lax.axis_index("c")
    base = wid * ROWS_PER_W

    pltpu.async_copy(cols_hbm, idx_v, isems[0]).wait()
    col_vecs = [idx_v[pl.ds(j * L, L)] for j in range(NG)]
    iota = lax.iota(jnp.int32, L)
    out_off = [j * L + iota for j in range(NG)]
    tail_mask = (NG - 1) * L + iota < K

    def in_slice(chunk):
        return x_hbm.at[pl.ds((base + chunk * R) * N_COLS, R * N_COLS)]

    def out_slice(chunk):
        return out_hbm.at[pl.ds((base + chunk * R) * K, R * K)]

    def compute(in_b, out_b):
        @pl.loop(0, R, unroll=4)
        def _row(r):
            rbase = r * N_COLS
            rk = r * K
            for j in range(NG):
                vals = plsc.load_gather(in_b, [rbase + col_vecs[j]])
                plsc.store_scatter(
                    out_b,
                    [rk + out_off[j]],
                    vals,
                    mask=tail_mask if j == NG - 1 else None,
                )

    for b in range(NBUF):
        pltpu.async_copy(in_slice(b), in_bufs[b], isems[b])

    @pl.loop(0, NCHUNK, step=NBUF)
    def _g(g):
        for b in range(NBUF):
            chunk = g + b
            pltpu.make_async_copy(in_slice(chunk), in_bufs[b], isems[b]).wait()

            @pl.when(chunk >= NBUF)
            def _wait_out():
                pltpu.make_async_copy(
                    out_bufs[b], out_slice(chunk), osems[b]
                ).wait()

            compute(in_bufs[b], out_bufs[b])
            pltpu.async_copy(out_bufs[b], out_slice(chunk), osems[b])

            @pl.when(chunk + NBUF < NCHUNK)
            def _next_in():
                pltpu.async_copy(in_slice(chunk + NBUF), in_bufs[b], isems[b])

    for b in range(NBUF):
        pltpu.make_async_copy(
            out_bufs[b], out_slice(NCHUNK - NBUF + b), osems[b]
        ).wait()


def kernel(x, column_indices):
    cols = jnp.asarray(column_indices, jnp.int32)
    cols_padded = jnp.concatenate([cols, jnp.zeros((KPAD - K,), jnp.int32)])
    out_flat = _sc_gather(x.reshape(-1), cols_padded)
    return out_flat.reshape(N_ROWS, K)


# speedup vs baseline: 2.3294x; 2.0609x over previous
import functools

import jax
import jax.numpy as jnp
from jax import lax
from jax.experimental import pallas as pl
from jax.experimental.pallas import tpu as pltpu
from jax.experimental.pallas import tpu_sc as plsc

N_ROWS = 16384
N_COLS = 1024
K = 100
L = 16
NG = (K + L - 1) // L
KPAD = NG * L
NC = 2
NS = 16
NW = NC * NS
ROWS_PER_W = N_ROWS // NW
R = 32
NCHUNK = ROWS_PER_W // R
NBUF = 2

_mesh = plsc.VectorSubcoreMesh(core_axis_name="c", subcore_axis_name="s")


@functools.partial(
    pl.kernel,
    out_type=jax.ShapeDtypeStruct((N_ROWS, K), jnp.float32),
    mesh=_mesh,
    scratch_types=[
        pltpu.VMEM((KPAD,), jnp.int32),
        [pltpu.VMEM((R, N_COLS), jnp.float32) for _ in range(NBUF)],
        [pltpu.VMEM((R, K), jnp.float32) for _ in range(NBUF)],
        [pltpu.SemaphoreType.DMA for _ in range(NBUF)],
        [pltpu.SemaphoreType.DMA for _ in range(NBUF)],
    ],
    compiler_params=pltpu.CompilerParams(
        use_tc_tiling_on_sc=True, needs_layout_passes=False
    ),
)
def _sc_gather(x_hbm, cols_hbm, out_hbm, idx_v, in_bufs, out_bufs, isems, osems):
    wid = lax.axis_index("s") * NC + lax.axis_index("c")
    base = wid * ROWS_PER_W

    pltpu.async_copy(cols_hbm, idx_v, isems[0]).wait()
    col_vecs = [idx_v[pl.ds(j * L, L)] for j in range(NG)]
    iota = lax.iota(jnp.int32, L)
    out_off = [j * L + iota for j in range(NG)]
    tail_mask = (NG - 1) * L + iota < K

    def in_slice(chunk):
        return x_hbm.at[pl.ds(base + chunk * R, R), :]

    def out_slice(chunk):
        return out_hbm.at[pl.ds(base + chunk * R, R), :]

    def compute(in_b, out_b):
        @pl.loop(0, R, unroll=4)
        def _row(r):
            row_splat = jnp.full((L,), r, jnp.int32)
            for j in range(NG):
                vals = plsc.load_gather(in_b, [row_splat, col_vecs[j]])
                plsc.store_scatter(
                    out_b,
                    [row_splat, out_off[j]],
                    vals,
                    mask=tail_mask if j == NG - 1 else None,
                )

    for b in range(NBUF):
        pltpu.async_copy(in_slice(b), in_bufs[b], isems[b])

    @pl.loop(0, NCHUNK, step=NBUF)
    def _g(g):
        for b in range(NBUF):
            chunk = g + b
            pltpu.make_async_copy(in_slice(chunk), in_bufs[b], isems[b]).wait()

            @pl.when(chunk >= NBUF)
            def _wait_out():
                pltpu.make_async_copy(
                    out_bufs[b], out_slice(chunk), osems[b]
                ).wait()

            compute(in_bufs[b], out_bufs[b])
            pltpu.async_copy(out_bufs[b], out_slice(chunk), osems[b])

            @pl.when(chunk + NBUF < NCHUNK)
            def _next_in():
                pltpu.async_copy(in_slice(chunk + NBUF), in_bufs[b], isems[b])

    for b in range(NBUF):
        pltpu.make_async_copy(
            out_bufs[b], out_slice(NCHUNK - NBUF + b), osems[b]
        ).wait()


def kernel(x, column_indices):
    cols = jnp.asarray(column_indices, jnp.int32)
    cols_padded = jnp.concatenate([cols, jnp.zeros((KPAD - K,), jnp.int32)])
    return _sc_gather(x, cols_padded)
